# Initial kernel scaffold; baseline (speedup 1.0000x reference)
import jax
import jax.numpy as jnp
from jax.experimental import pallas as pl


def kernel(input_ids, params):
    raise NotImplementedError("write your pallas kernel here")


# speedup vs baseline: 1.0829x; 1.0829x over previous
import functools

import jax
import jax.numpy as jnp
from jax import lax
from jax.experimental import pallas as pl
from jax.experimental.pallas import tpu as pltpu
from jax.experimental.pallas import tpu_sc as plsc

VOCAB = 100000
SEQ = 2048
BATCH = 2
D = 1024
H = 8
HD = 128
FF = 4096
NSH = 6
NE = 7
TOPK = 2
T = BATCH * SEQ

BT = 512
NT = T // BT
NSB = SEQ // BT
NF = FF // 1024
NEG_INF = -1e30
HI = jax.lax.Precision.HIGHEST


def _layernorm(x, g, b):
    m = jnp.mean(x, axis=-1, keepdims=True)
    v = jnp.mean((x - m) ** 2, axis=-1, keepdims=True)
    return (x - m) / jnp.sqrt(v + 1e-5) * g + b


def _slabsum(xt):
    acc = xt[0:8]
    for kk in range(1, D // 8):
        acc = acc + xt[8 * kk:8 * (kk + 1)]
    u = acc[0:4] + acc[4:8]
    w = u[0:2] + u[2:4]
    return w[0:1] + w[1:2]


def _statsA_body(xT_ref, posT_ref, m_ref, v_ref):
    xt = xT_ref[...] + posT_ref[...]
    m = _slabsum(xt) / float(D)
    d = xt - m
    v = _slabsum(d * d) / float(D)
    m_ref[...] = m
    v_ref[...] = v


def _statsB_body(xT_ref, m_ref, v_ref):
    xt = xT_ref[...]
    m = _slabsum(xt) / float(D)
    d = xt - m
    v = _slabsum(d * d) / float(D)
    m_ref[...] = m
    v_ref[...] = v


def _stats_call(body, *args):
    n_in = len(args)
    return pl.pallas_call(
        body,
        grid=(NT,),
        in_specs=[pl.BlockSpec((D, BT), lambda t: (0, t))] +
                 [pl.BlockSpec((D, BT), lambda t: (0, t % NSB))] * (n_in - 1),
        out_specs=[pl.BlockSpec((1, BT), lambda t: (0, t)),
                   pl.BlockSpec((1, BT), lambda t: (0, t))],
        out_shape=[jax.ShapeDtypeStruct((1, T), jnp.float32)] * 2,
    )(*args)


def _sc_gather(table, idx, n_rows, d):
    info = plsc.get_sparse_core_info()
    nc, ns = info.num_cores, info.num_subcores
    nw = nc * ns
    per_w = n_rows // nw
    chunk = min(per_w, 64)
    nchunk = per_w // chunk
    mesh = plsc.VectorSubcoreMesh(core_axis_name="c", subcore_axis_name="s")

    @functools.partial(
        pl.kernel,
        mesh=mesh,
        out_type=jax.ShapeDtypeStruct((n_rows, d), jnp.float32),
        scratch_types=[
            pltpu.VMEM((chunk,), jnp.int32),
            pltpu.VMEM((chunk, d), jnp.float32),
            pltpu.SemaphoreType.DMA,
        ],
    )
    def k(table_hbm, idx_hbm, out_hbm, idx_v, rows_v, sem):
        wid = lax.axis_index("s") * nc + lax.axis_index("c")
        base = wid * per_w
        for c in range(nchunk):
            off = base + c * chunk
            pltpu.sync_copy(idx_hbm.at[pl.ds(off, chunk)], idx_v)
            pltpu.async_copy(table_hbm.at[idx_v], rows_v, sem).wait()
            pltpu.sync_copy(rows_v, out_hbm.at[pl.ds(off, chunk)])

    return k(table, idx)


def _k2_body(xe_ref, pos_ref, wq_ref, wk_ref, wv_ref, bq_ref, bk_ref, bv_ref,
             g_ref, b_ref, cos_ref, sin_ref, m_ref, vv_ref,
             x_ref, q_ref, k_ref, v_ref):
    x = xe_ref[...] + pos_ref[...]
    x_ref[...] = x
    h = (x - m_ref[...]) / jnp.sqrt(vv_ref[...] + 1e-5) * g_ref[...] + b_ref[...]
    cos = cos_ref[...]
    sin = sin_ref[...]

    hb = h.astype(jnp.bfloat16)

    def proj_rope(w_ref, bias_ref, do_rope):
        y = jnp.dot(hb, w_ref[...], preferred_element_type=jnp.float32)
        y = y + bias_ref[...]
        if not do_rope:
            return y
        parts = []
        for hh in range(H):
            y1 = y[:, hh * HD:hh * HD + HD // 2]
            y2 = y[:, hh * HD + HD // 2:(hh + 1) * HD]
            parts.append(y1 * cos - y2 * sin)
            parts.append(y1 * sin + y2 * cos)
        return jnp.concatenate(parts, axis=1)

    q_ref[...] = proj_rope(wq_ref, bq_ref, True)
    k_ref[...] = proj_rope(wk_ref, bk_ref, True)
    v_ref[...] = jnp.dot(hb, wv_ref[...],
                         preferred_element_type=jnp.float32) + bv_ref[...]


def _k3_body(q_ref, k_ref, v_ref, attn_ref, sa_ref):
    q = q_ref[...]
    k = k_ref[...]
    scores = jax.lax.dot_general(
        q, k, (((1,), (1,)), ((), ())),
        preferred_element_type=jnp.float32) * (1.0 / (HD ** 0.5))
    m = jnp.max(scores, axis=1, keepdims=True)
    e = jnp.exp(scores - m)
    p = e / jnp.sum(e, axis=1, keepdims=True)
    attn_ref[0, 0] = p
    sa_ref[...] = jnp.dot(p.astype(jnp.bfloat16), v_ref[...],
                          preferred_element_type=jnp.float32)


def _k4a_body(x_ref, sa_ref, wo_ref, bo_ref, x2_ref):
    sa2 = jnp.dot(sa_ref[...].astype(jnp.bfloat16), wo_ref[...],
                  preferred_element_type=jnp.float32) + bo_ref[...]
    x2_ref[...] = x_ref[...] + sa2


def _k4b_body(x2_ref, m_ref, vv_ref, g_ref, b_ref, rw_ref, rb_ref,
              masks_ref, me_ref, ce_ref):
    step = pl.program_id(0)
    x2 = x2_ref[...]
    h2 = (x2 - m_ref[...]) / jnp.sqrt(vv_ref[...] + 1e-5) * g_ref[...] + b_ref[...]
    rl = jnp.dot(h2, rw_ref[...], precision=HI,
                 preferred_element_type=jnp.float32) + rb_ref[...]
    lane = jax.lax.broadcasted_iota(jnp.int32, (BT, 128), 1)
    valid = lane < NE
    rl = jnp.where(valid, rl, NEG_INF)
    mx = jnp.max(rl, axis=1, keepdims=True)
    ex = jnp.exp(rl - mx)
    ex = jnp.where(valid, ex, 0.0)
    p = ex / jnp.sum(ex, axis=1, keepdims=True)

    psel = jnp.where(valid, p, -1.0)
    i1 = jnp.argmax(psel, axis=1, keepdims=True)
    oh1 = lane == i1
    v1 = jnp.max(psel, axis=1, keepdims=True)
    psel2 = jnp.where(oh1, -1.0, psel)
    i2 = jnp.argmax(psel2, axis=1, keepdims=True)
    oh2 = lane == i2
    v2 = jnp.max(psel2, axis=1, keepdims=True)
    masks = jnp.where(oh1, v1, 0.0) + jnp.where(oh2, v2, 0.0)
    masks_ref[...] = masks

    me_part = jnp.sum(p, axis=0, keepdims=True)
    ce_part = jnp.sum((masks > 0).astype(jnp.float32), axis=0, keepdims=True)

    @pl.when(step == 0)
    def _():
        me_ref[...] = jnp.zeros_like(me_ref)
        ce_ref[...] = jnp.zeros_like(ce_ref)

    me_ref[...] += me_part
    ce_ref[...] += ce_part


def _k5_body(x_ref, w1_ref, b1_ref, w2_ref, b2_ref, masks_ref, acc_ref):
    e = pl.program_id(1)
    f = pl.program_id(2)
    lane = jax.lax.broadcasted_iota(jnp.int32, (BT, 128), 1)
    gate = jnp.sum(jnp.where(lane == e, masks_ref[...], 0.0),
                   axis=1, keepdims=True)

    @pl.when(jnp.logical_and(e == 0, f == 0))
    def _():
        acc_ref[...] = jnp.zeros_like(acc_ref)

    hfull = jnp.dot(x_ref[...], w1_ref[0],
                    preferred_element_type=jnp.float32) + b1_ref[0]
    hact = jax.nn.gelu(hfull).astype(jnp.bfloat16)
    part = jnp.dot(hact, w2_ref[0], preferred_element_type=jnp.float32)

    @pl.when(f == 0)
    def _():
        acc_ref[...] += gate * b2_ref[0]

    acc_ref[...] += gate * part


def _k6_body(acc_ref, g_ref, b_ref, cw_ref, cb_ref, me_ref, ce_ref,
             eo_ref, fv_ref, cls_ref, loss_ref):
    step = pl.program_id(0)
    eo = _layernorm(acc_ref[...], g_ref[...], b_ref[...])
    eo_ref[...] = eo

    @pl.when(step == 0)
    def _():
        fv_ref[...] = jnp.zeros_like(fv_ref)

    b_id = step // NSB
    rowsum = jnp.sum(eo, axis=0, keepdims=True)
    brow = jax.lax.broadcasted_iota(jnp.int32, (BATCH, D), 0)
    fv_ref[...] += jnp.where(brow == b_id, rowsum, 0.0)

    @pl.when(step == NT - 1)
    def _():
        fv = fv_ref[...] * (1.0 / SEQ)
        fv_ref[...] = fv
        cls_ref[...] = jnp.dot(fv, cw_ref[...],
                               preferred_element_type=jnp.float32) + cb_ref[...]
        me = me_ref[...] * (1.0 / T)
        ce = ce_ref[...] * (1.0 / T)
        loss_ref[...] = NE * jnp.sum(me * ce, keepdims=True)


def kernel(input_ids, params):
    p = params
    f32 = jnp.float32

    ids = input_ids.reshape(T).astype(jnp.int32)
    xe = _sc_gather(p['embedding'], ids, T, D)

    half = HD // 2
    freqs = 1.0 / (10000.0 ** (jnp.arange(half, dtype=f32) / half))
    ang = jnp.arange(SEQ, dtype=f32)[:, None] * freqs[None, :]
    cos = jnp.cos(ang)
    sin = jnp.sin(ang)

    row = lambda a: a.reshape(1, -1)

    xeT = jnp.transpose(xe)
    posT = jnp.transpose(p['pos'])
    m1, v1 = _stats_call(_statsA_body, xeT, posT)
    m1c, v1c = jnp.transpose(m1), jnp.transpose(v1)

    x, q, k, v = pl.pallas_call(
        _k2_body,
        grid=(NT,),
        in_specs=[
            pl.BlockSpec((BT, D), lambda t: (t, 0)),
            pl.BlockSpec((BT, D), lambda t: (t % NSB, 0)),
            pl.BlockSpec((D, D), lambda t: (0, 0)),
            pl.BlockSpec((D, D), lambda t: (0, 0)),
            pl.BlockSpec((D, D), lambda t: (0, 0)),
            pl.BlockSpec((1, D), lambda t: (0, 0)),
            pl.BlockSpec((1, D), lambda t: (0, 0)),
            pl.BlockSpec((1, D), lambda t: (0, 0)),
            pl.BlockSpec((1, D), lambda t: (0, 0)),
            pl.BlockSpec((1, D), lambda t: (0, 0)),
            pl.BlockSpec((BT, half), lambda t: (t % NSB, 0)),
            pl.BlockSpec((BT, half), lambda t: (t % NSB, 0)),
            pl.BlockSpec((BT, 1), lambda t: (t, 0)),
            pl.BlockSpec((BT, 1), lambda t: (t, 0)),
        ],
        out_specs=[
            pl.BlockSpec((BT, D), lambda t: (t, 0)),
            pl.BlockSpec((BT, D), lambda t: (t, 0)),
            pl.BlockSpec((BT, D), lambda t: (t, 0)),
            pl.BlockSpec((BT, D), lambda t: (t, 0)),
        ],
        out_shape=[jax.ShapeDtypeStruct((T, D), f32)] * 4,
    )(xe, p['pos'], p['Wq'], p['Wk'], p['Wv'],
      row(p['bq']), row(p['bk']), row(p['bv']),
      row(p['ln1_g']), row(p['ln1_b']), cos, sin, m1c, v1c)

    attn, sa = pl.pallas_call(
        _k3_body,
        grid=(BATCH, H, NSB),
        in_specs=[
            pl.BlockSpec((BT, HD), lambda b, h, qi: (b * NSB + qi, h)),
            pl.BlockSpec((SEQ, HD), lambda b, h, qi: (b, h)),
            pl.BlockSpec((SEQ, HD), lambda b, h, qi: (b, h)),
        ],
        out_specs=[
            pl.BlockSpec((1, 1, BT, SEQ), lambda b, h, qi: (b, h, qi, 0)),
            pl.BlockSpec((BT, HD), lambda b, h, qi: (b * NSB + qi, h)),
        ],
        out_shape=[
            jax.ShapeDtypeStruct((BATCH, H, SEQ, SEQ), f32),
            jax.ShapeDtypeStruct((T, D), f32),
        ],
    )(q, k, v)

    x2 = pl.pallas_call(
        _k4a_body,
        grid=(NT,),
        in_specs=[
            pl.BlockSpec((BT, D), lambda t: (t, 0)),
            pl.BlockSpec((BT, D), lambda t: (t, 0)),
            pl.BlockSpec((D, D), lambda t: (0, 0)),
            pl.BlockSpec((1, D), lambda t: (0, 0)),
        ],
        out_specs=pl.BlockSpec((BT, D), lambda t: (t, 0)),
        out_shape=jax.ShapeDtypeStruct((T, D), f32),
    )(x, sa, p['Wo'], row(p['bo']))

    m2, v2 = _stats_call(_statsB_body, jnp.transpose(x2))
    m2c, v2c = jnp.transpose(m2), jnp.transpose(v2)

    rw_pad = jnp.zeros((D, 128), f32).at[:, :NE].set(p['router_W'])
    rb_pad = jnp.zeros((1, 128), f32).at[0, :NE].set(p['router_b'])
    masks_pad, me_sum, ce_sum = pl.pallas_call(
        _k4b_body,
        grid=(NT,),
        in_specs=[
            pl.BlockSpec((BT, D), lambda t: (t, 0)),
            pl.BlockSpec((BT, 1), lambda t: (t, 0)),
            pl.BlockSpec((BT, 1), lambda t: (t, 0)),
            pl.BlockSpec((1, D), lambda t: (0, 0)),
            pl.BlockSpec((1, D), lambda t: (0, 0)),
            pl.BlockSpec((D, 128), lambda t: (0, 0)),
            pl.BlockSpec((1, 128), lambda t: (0, 0)),
        ],
        out_specs=[
            pl.BlockSpec((BT, 128), lambda t: (t, 0)),
            pl.BlockSpec((1, 128), lambda t: (0, 0)),
            pl.BlockSpec((1, 128), lambda t: (0, 0)),
        ],
        out_shape=[
            jax.ShapeDtypeStruct((T, 128), f32),
            jax.ShapeDtypeStruct((1, 128), f32),
            jax.ShapeDtypeStruct((1, 128), f32),
        ],
    )(x2, m2c, v2c, row(p['ln2_g']), row(p['ln2_b']), rw_pad, rb_pad)

    w1 = jnp.concatenate([p['shared_W1'], p['text_W1'][None]], axis=0)
    b1 = jnp.concatenate([p['shared_b1'], p['text_b1'][None]],
                         axis=0).reshape(NE * NF, 1, 1024)
    w2 = jnp.concatenate([p['shared_W2'], p['text_W2'][None]], axis=0)
    b2 = jnp.concatenate([p['shared_b2'], p['text_b2'][None]],
                         axis=0).reshape(NE, 1, D)
    acc = pl.pallas_call(
        _k5_body,
        grid=(NT, NE, NF),
        in_specs=[
            pl.BlockSpec((BT, D), lambda t, e, f: (t, 0)),
            pl.BlockSpec((1, D, 1024), lambda t, e, f: (e, 0, f)),
            pl.BlockSpec((1, 1, 1024), lambda t, e, f: (e * NF + f, 0, 0)),
            pl.BlockSpec((1, 1024, D), lambda t, e, f: (e, f, 0)),
            pl.BlockSpec((1, 1, D), lambda t, e, f: (e, 0, 0)),
            pl.BlockSpec((BT, 128), lambda t, e, f: (t, 0)),
        ],
        out_specs=pl.BlockSpec((BT, D), lambda t, e, f: (t, 0)),
        out_shape=jax.ShapeDtypeStruct((T, D), f32),
    )(x2, w1, b1, w2, b2, masks_pad)

    eo, fv, cls, loss = pl.pallas_call(
        _k6_body,
        grid=(NT,),
        in_specs=[
            pl.BlockSpec((BT, D), lambda t: (t, 0)),
            pl.BlockSpec((1, D), lambda t: (0, 0)),
            pl.BlockSpec((1, D), lambda t: (0, 0)),
            pl.BlockSpec((D, D), lambda t: (0, 0)),
            pl.BlockSpec((1, D), lambda t: (0, 0)),
            pl.BlockSpec((1, 128), lambda t: (0, 0)),
            pl.BlockSpec((1, 128), lambda t: (0, 0)),
        ],
        out_specs=[
            pl.BlockSpec((BT, D), lambda t: (t, 0)),
            pl.BlockSpec((BATCH, D), lambda t: (0, 0)),
            pl.BlockSpec((BATCH, D), lambda t: (0, 0)),
            pl.BlockSpec((1, 1), lambda t: (0, 0)),
        ],
        out_shape=[
            jax.ShapeDtypeStruct((T, D), f32),
            jax.ShapeDtypeStruct((BATCH, D), f32),
            jax.ShapeDtypeStruct((BATCH, D), f32),
            jax.ShapeDtypeStruct((1, 1), f32),
        ],
    )(acc, row(p['ln3_g']), row(p['ln3_b']), p['cls_W'], row(p['cls_b']),
      me_sum, ce_sum)

    masks = masks_pad[:, :NE].reshape(BATCH, SEQ, NE)
    expert_outputs = eo.reshape(BATCH, SEQ, D)
    return fv, cls, loss[0, 0], attn, expert_outputs, masks


# speedup vs baseline: 1.1215x; 1.0356x over previous
import functools

import jax
import jax.numpy as jnp
from jax import lax
from jax.experimental import pallas as pl
from jax.experimental.pallas import tpu as pltpu
from jax.experimental.pallas import tpu_sc as plsc

VOCAB = 100000
SEQ = 2048
BATCH = 2
D = 1024
H = 8
HD = 128
FF = 4096
NSH = 6
NE = 7
TOPK = 2
T = BATCH * SEQ

BT = 512
NT = T // BT
NSB = SEQ // BT
NF = FF // 1024
NEG_INF = -1e30
HI = jax.lax.Precision.HIGHEST


def _layernorm(x, g, b):
    m = jnp.mean(x, axis=-1, keepdims=True)
    v = jnp.mean((x - m) ** 2, axis=-1, keepdims=True)
    return (x - m) / jnp.sqrt(v + 1e-5) * g + b


def _slabsum(xt):
    acc = xt[0:8]
    for kk in range(1, D // 8):
        acc = acc + xt[8 * kk:8 * (kk + 1)]
    u = acc[0:4] + acc[4:8]
    w = u[0:2] + u[2:4]
    return w[0:1] + w[1:2]


def _statsA_body(xT_ref, posT_ref, m_ref, v_ref):
    xt = xT_ref[...] + posT_ref[...]
    m = _slabsum(xt) / float(D)
    d = xt - m
    v = _slabsum(d * d) / float(D)
    m_ref[...] = m
    v_ref[...] = v


def _statsB_body(xT_ref, m_ref, v_ref):
    xt = xT_ref[...]
    m = _slabsum(xt) / float(D)
    d = xt - m
    v = _slabsum(d * d) / float(D)
    m_ref[...] = m
    v_ref[...] = v


def _stats_call(body, *args):
    n_in = len(args)
    return pl.pallas_call(
        body,
        grid=(NT,),
        in_specs=[pl.BlockSpec((D, BT), lambda t: (0, t))] +
                 [pl.BlockSpec((D, BT), lambda t: (0, t % NSB))] * (n_in - 1),
        out_specs=[pl.BlockSpec((1, BT), lambda t: (0, t)),
                   pl.BlockSpec((1, BT), lambda t: (0, t))],
        out_shape=[jax.ShapeDtypeStruct((1, T), jnp.float32)] * 2,
    )(*args)


def _sc_gather(table, idx, n_rows, d):
    info = plsc.get_sparse_core_info()
    nc, ns = info.num_cores, info.num_subcores
    nw = nc * ns
    per_w = n_rows // nw
    chunk = min(per_w, 64)
    nchunk = per_w // chunk
    mesh = plsc.VectorSubcoreMesh(core_axis_name="c", subcore_axis_name="s")

    @functools.partial(
        pl.kernel,
        mesh=mesh,
        out_type=jax.ShapeDtypeStruct((n_rows, d), jnp.float32),
        scratch_types=[
            pltpu.VMEM((chunk,), jnp.int32),
            pltpu.VMEM((chunk, d), jnp.float32),
            pltpu.SemaphoreType.DMA,
        ],
    )
    def k(table_hbm, idx_hbm, out_hbm, idx_v, rows_v, sem):
        wid = lax.axis_index("s") * nc + lax.axis_index("c")
        base = wid * per_w
        for c in range(nchunk):
            off = base + c * chunk
            pltpu.sync_copy(idx_hbm.at[pl.ds(off, chunk)], idx_v)
            pltpu.async_copy(table_hbm.at[idx_v], rows_v, sem).wait()
            pltpu.sync_copy(rows_v, out_hbm.at[pl.ds(off, chunk)])

    return k(table, idx)


def _k2_body(xe_ref, pos_ref, wq_ref, wk_ref, wv_ref, bq_ref, bk_ref, bv_ref,
             g_ref, b_ref, cos_ref, sin_ref, m_ref, vv_ref,
             x_ref, q_ref, k_ref, v_ref):
    x = xe_ref[...] + pos_ref[...]
    x_ref[...] = x
    h = (x - m_ref[...]) / jnp.sqrt(vv_ref[...] + 1e-5) * g_ref[...] + b_ref[...]
    cos = cos_ref[...]
    sin = sin_ref[...]

    hb = h.astype(jnp.bfloat16)

    def proj_rope(w_ref, bias_ref, do_rope):
        y = jnp.dot(hb, w_ref[...], preferred_element_type=jnp.float32)
        y = y + bias_ref[...]
        if not do_rope:
            return y
        parts = []
        for hh in range(H):
            y1 = y[:, hh * HD:hh * HD + HD // 2]
            y2 = y[:, hh * HD + HD // 2:(hh + 1) * HD]
            parts.append(y1 * cos - y2 * sin)
            parts.append(y1 * sin + y2 * cos)
        return jnp.concatenate(parts, axis=1)

    q_ref[...] = proj_rope(wq_ref, bq_ref, True)
    k_ref[...] = proj_rope(wk_ref, bk_ref, True)
    v_ref[...] = jnp.dot(hb, wv_ref[...],
                         preferred_element_type=jnp.float32) + bv_ref[...]


def _k3_body(q_ref, k_ref, v_ref, attn_ref, sa_ref):
    q = q_ref[...]
    k = k_ref[...]
    scores = jax.lax.dot_general(
        q, k, (((1,), (1,)), ((), ())),
        preferred_element_type=jnp.float32) * (1.0 / (HD ** 0.5))
    m = jnp.max(scores, axis=1, keepdims=True)
    e = jnp.exp(scores - m)
    p = e / jnp.sum(e, axis=1, keepdims=True)
    attn_ref[0, 0] = p
    CK = 1024
    m_run = jnp.full((BT, 1), -jnp.inf, jnp.float32)
    l_run = jnp.zeros((BT, 1), jnp.float32)
    acc = jnp.zeros((BT, HD), jnp.float32)
    for c in range(SEQ // CK):
        s_c = scores[:, c * CK:(c + 1) * CK]
        bm = jnp.max(s_c, axis=1, keepdims=True)
        m_new = jnp.maximum(m_run, bm)
        corr = jnp.where(m_run == m_new, 0.0, m_run - m_new)
        ec = jnp.exp(corr)
        eb = jnp.exp(s_c - m_new)
        l_run = ec * l_run + jnp.sum(eb, axis=1, keepdims=True)
        acc = jnp.dot(eb, v_ref[c * CK:(c + 1) * CK, :],
                      preferred_element_type=jnp.float32) + ec * acc
        m_run = m_new
    sa_ref[...] = acc * (1.0 / l_run)


def _k4a_body(x_ref, sa_ref, wo_ref, bo_ref, x2_ref):
    sa2 = jnp.dot(sa_ref[...].astype(jnp.bfloat16), wo_ref[...],
                  preferred_element_type=jnp.float32) + bo_ref[...]
    x2_ref[...] = x_ref[...] + sa2


def _k4b_body(x2_ref, m_ref, vv_ref, g_ref, b_ref, rw_ref, rb_ref,
              masks_ref, me_ref, ce_ref):
    step = pl.program_id(0)
    x2 = x2_ref[...]
    h2 = (x2 - m_ref[...]) / jnp.sqrt(vv_ref[...] + 1e-5) * g_ref[...] + b_ref[...]
    rl = jnp.dot(h2, rw_ref[...], precision=HI,
                 preferred_element_type=jnp.float32) + rb_ref[...]
    lane = jax.lax.broadcasted_iota(jnp.int32, (BT, 128), 1)
    valid = lane < NE
    rl = jnp.where(valid, rl, NEG_INF)
    mx = jnp.max(rl, axis=1, keepdims=True)
    ex = jnp.exp(rl - mx)
    ex = jnp.where(valid, ex, 0.0)
    p = ex / jnp.sum(ex, axis=1, keepdims=True)

    psel = jnp.where(valid, p, -1.0)
    i1 = jnp.argmax(psel, axis=1, keepdims=True)
    oh1 = lane == i1
    v1 = jnp.max(psel, axis=1, keepdims=True)
    psel2 = jnp.where(oh1, -1.0, psel)
    i2 = jnp.argmax(psel2, axis=1, keepdims=True)
    oh2 = lane == i2
    v2 = jnp.max(psel2, axis=1, keepdims=True)
    masks = jnp.where(oh1, v1, 0.0) + jnp.where(oh2, v2, 0.0)
    masks_ref[...] = masks

    me_part = jnp.sum(p, axis=0, keepdims=True)
    ce_part = jnp.sum((masks > 0).astype(jnp.float32), axis=0, keepdims=True)

    @pl.when(step == 0)
    def _():
        me_ref[...] = jnp.zeros_like(me_ref)
        ce_ref[...] = jnp.zeros_like(ce_ref)

    me_ref[...] += me_part
    ce_ref[...] += ce_part


def _k4c_body(x2T_ref, m_ref, vv_ref, gT_ref, bT_ref, rw_ref, rb_ref,
              masksT_ref, me_ref, ce_ref):
    step = pl.program_id(0)
    h2 = (x2T_ref[...] - m_ref[...]) / jnp.sqrt(vv_ref[...] + 1e-5) \
        * gT_ref[...] + bT_ref[...]
    rls = []
    for e in range(NE):
        w = rw_ref[:, e:e + 1]
        rls.append(_slabsum(h2 * w))
    rls.append(jnp.full((1, BT), NEG_INF, jnp.float32))
    rl = jnp.concatenate(rls, axis=0) + rb_ref[...]
    srow = jax.lax.broadcasted_iota(jnp.int32, (8, BT), 0)
    mx = jnp.max(rl, axis=0, keepdims=True)
    ex = jnp.exp(rl - mx)
    ex = jnp.where(srow < NE, ex, 0.0)
    pp = ex / jnp.sum(ex, axis=0, keepdims=True)
    psel = jnp.where(srow < NE, pp, -1.0)
    i1 = jnp.argmax(psel, axis=0, keepdims=True)
    oh1 = srow == i1
    v1 = jnp.max(psel, axis=0, keepdims=True)
    psel2 = jnp.where(oh1, -1.0, psel)
    i2 = jnp.argmax(psel2, axis=0, keepdims=True)
    oh2 = srow == i2
    v2 = jnp.max(psel2, axis=0, keepdims=True)
    masks = jnp.where(oh1, v1, 0.0) + jnp.where(oh2, v2, 0.0)
    masksT_ref[...] = masks
    me_part = jnp.sum(pp, axis=1, keepdims=True)
    ce_part = jnp.sum((masks > 0).astype(jnp.float32), axis=1, keepdims=True)

    @pl.when(step == 0)
    def _():
        me_ref[...] = jnp.zeros_like(me_ref)
        ce_ref[...] = jnp.zeros_like(ce_ref)

    me_ref[...] += me_part
    ce_ref[...] += ce_part


def _k5_body(x_ref, w1_ref, b1_ref, w2_ref, b2_ref, masks_ref, acc_ref):
    e = pl.program_id(1)
    f = pl.program_id(2)
    lane = jax.lax.broadcasted_iota(jnp.int32, (BT, 8), 1)
    gate = jnp.sum(jnp.where(lane == e, masks_ref[...], 0.0),
                   axis=1, keepdims=True)

    @pl.when(jnp.logical_and(e == 0, f == 0))
    def _():
        acc_ref[...] = jnp.zeros_like(acc_ref)

    hfull = jnp.dot(x_ref[...], w1_ref[0],
                    preferred_element_type=jnp.float32) + b1_ref[0]
    hact = jax.nn.gelu(hfull).astype(jnp.bfloat16)
    part = jnp.dot(hact, w2_ref[0], preferred_element_type=jnp.float32)

    @pl.when(f == 0)
    def _():
        acc_ref[...] += gate * b2_ref[0]

    acc_ref[...] += gate * part


def _k6_body(acc_ref, g_ref, b_ref, cw_ref, cb_ref, me_ref, ce_ref,
             eo_ref, fv_ref, cls_ref, loss_ref):
    step = pl.program_id(0)
    eo = _layernorm(acc_ref[...], g_ref[...], b_ref[...])
    eo_ref[...] = eo

    @pl.when(step == 0)
    def _():
        fv_ref[...] = jnp.zeros_like(fv_ref)

    b_id = step // NSB
    rowsum = jnp.sum(eo, axis=0, keepdims=True)
    brow = jax.lax.broadcasted_iota(jnp.int32, (BATCH, D), 0)
    fv_ref[...] += jnp.where(brow == b_id, rowsum, 0.0)

    @pl.when(step == NT - 1)
    def _():
        fv = fv_ref[...] * (1.0 / SEQ)
        fv_ref[...] = fv
        cls_ref[...] = jnp.dot(fv, cw_ref[...],
                               preferred_element_type=jnp.float32) + cb_ref[...]
        me = me_ref[...] * (1.0 / T)
        ce = ce_ref[...] * (1.0 / T)
        loss_ref[...] = NE * jnp.sum(me * ce, keepdims=True)


def kernel(input_ids, params):
    p = params
    f32 = jnp.float32

    ids = input_ids.reshape(T).astype(jnp.int32)
    xe = _sc_gather(p['embedding'], ids, T, D)

    half = HD // 2
    freqs = 1.0 / (10000.0 ** (jnp.arange(half, dtype=f32) / half))
    ang = jnp.arange(SEQ, dtype=f32)[:, None] * freqs[None, :]
    cos = jnp.cos(ang)
    sin = jnp.sin(ang)

    row = lambda a: a.reshape(1, -1)

    xeT = jnp.transpose(xe)
    posT = jnp.transpose(p['pos'])
    m1, v1 = _stats_call(_statsA_body, xeT, posT)
    m1c, v1c = jnp.transpose(m1), jnp.transpose(v1)

    x, q, k, v = pl.pallas_call(
        _k2_body,
        grid=(NT,),
        in_specs=[
            pl.BlockSpec((BT, D), lambda t: (t, 0)),
            pl.BlockSpec((BT, D), lambda t: (t % NSB, 0)),
            pl.BlockSpec((D, D), lambda t: (0, 0)),
            pl.BlockSpec((D, D), lambda t: (0, 0)),
            pl.BlockSpec((D, D), lambda t: (0, 0)),
            pl.BlockSpec((1, D), lambda t: (0, 0)),
            pl.BlockSpec((1, D), lambda t: (0, 0)),
            pl.BlockSpec((1, D), lambda t: (0, 0)),
            pl.BlockSpec((1, D), lambda t: (0, 0)),
            pl.BlockSpec((1, D), lambda t: (0, 0)),
            pl.BlockSpec((BT, half), lambda t: (t % NSB, 0)),
            pl.BlockSpec((BT, half), lambda t: (t % NSB, 0)),
            pl.BlockSpec((BT, 1), lambda t: (t, 0)),
            pl.BlockSpec((BT, 1), lambda t: (t, 0)),
        ],
        out_specs=[
            pl.BlockSpec((BT, D), lambda t: (t, 0)),
            pl.BlockSpec((BT, D), lambda t: (t, 0)),
            pl.BlockSpec((BT, D), lambda t: (t, 0)),
            pl.BlockSpec((BT, D), lambda t: (t, 0)),
        ],
        out_shape=[jax.ShapeDtypeStruct((T, D), f32)] * 4,
    )(xe, p['pos'], p['Wq'], p['Wk'], p['Wv'],
      row(p['bq']), row(p['bk']), row(p['bv']),
      row(p['ln1_g']), row(p['ln1_b']), cos, sin, m1c, v1c)

    attn, sa = pl.pallas_call(
        _k3_body,
        grid=(BATCH, H, NSB),
        in_specs=[
            pl.BlockSpec((BT, HD), lambda b, h, qi: (b * NSB + qi, h)),
            pl.BlockSpec((SEQ, HD), lambda b, h, qi: (b, h)),
            pl.BlockSpec((SEQ, HD), lambda b, h, qi: (b, h)),
        ],
        out_specs=[
            pl.BlockSpec((1, 1, BT, SEQ), lambda b, h, qi: (b, h, qi, 0)),
            pl.BlockSpec((BT, HD), lambda b, h, qi: (b * NSB + qi, h)),
        ],
        out_shape=[
            jax.ShapeDtypeStruct((BATCH, H, SEQ, SEQ), f32),
            jax.ShapeDtypeStruct((T, D), f32),
        ],
    )(q, k, v)

    x2 = pl.pallas_call(
        _k4a_body,
        grid=(NT,),
        in_specs=[
            pl.BlockSpec((BT, D), lambda t: (t, 0)),
            pl.BlockSpec((BT, D), lambda t: (t, 0)),
            pl.BlockSpec((D, D), lambda t: (0, 0)),
            pl.BlockSpec((1, D), lambda t: (0, 0)),
        ],
        out_specs=pl.BlockSpec((BT, D), lambda t: (t, 0)),
        out_shape=jax.ShapeDtypeStruct((T, D), f32),
    )(x, sa, p['Wo'], row(p['bo']))

    x2T = jnp.transpose(x2)
    m2, v2 = _stats_call(_statsB_body, x2T)

    rw_pad = jnp.zeros((D, 128), f32).at[:, :NE].set(p['router_W'])
    rb_col = jnp.zeros((8, 1), f32).at[:NE, 0].set(p['router_b'])
    masksT, me_sum, ce_sum = pl.pallas_call(
        _k4c_body,
        grid=(NT,),
        in_specs=[
            pl.BlockSpec((D, BT), lambda t: (0, t)),
            pl.BlockSpec((1, BT), lambda t: (0, t)),
            pl.BlockSpec((1, BT), lambda t: (0, t)),
            pl.BlockSpec((D, 1), lambda t: (0, 0)),
            pl.BlockSpec((D, 1), lambda t: (0, 0)),
            pl.BlockSpec((D, 128), lambda t: (0, 0)),
            pl.BlockSpec((8, 1), lambda t: (0, 0)),
        ],
        out_specs=[
            pl.BlockSpec((8, BT), lambda t: (0, t)),
            pl.BlockSpec((8, 1), lambda t: (0, 0)),
            pl.BlockSpec((8, 1), lambda t: (0, 0)),
        ],
        out_shape=[
            jax.ShapeDtypeStruct((8, T), f32),
            jax.ShapeDtypeStruct((8, 1), f32),
            jax.ShapeDtypeStruct((8, 1), f32),
        ],
    )(x2T, m2, v2, p['ln2_g'].reshape(D, 1), p['ln2_b'].reshape(D, 1),
      rw_pad, rb_col)
    masks_col = jnp.transpose(masksT)

    w1 = jnp.concatenate([p['shared_W1'], p['text_W1'][None]], axis=0)
    b1 = jnp.concatenate([p['shared_b1'], p['text_b1'][None]],
                         axis=0).reshape(NE * NF, 1, 1024)
    w2 = jnp.concatenate([p['shared_W2'], p['text_W2'][None]], axis=0)
    b2 = jnp.concatenate([p['shared_b2'], p['text_b2'][None]],
                         axis=0).reshape(NE, 1, D)
    acc = pl.pallas_call(
        _k5_body,
        grid=(NT, NE, NF),
        in_specs=[
            pl.BlockSpec((BT, D), lambda t, e, f: (t, 0)),
            pl.BlockSpec((1, D, 1024), lambda t, e, f: (e, 0, f)),
            pl.BlockSpec((1, 1, 1024), lambda t, e, f: (e * NF + f, 0, 0)),
            pl.BlockSpec((1, 1024, D), lambda t, e, f: (e, f, 0)),
            pl.BlockSpec((1, 1, D), lambda t, e, f: (e, 0, 0)),
            pl.BlockSpec((BT, 8), lambda t, e, f: (t, 0)),
        ],
        out_specs=pl.BlockSpec((BT, D), lambda t, e, f: (t, 0)),
        out_shape=jax.ShapeDtypeStruct((T, D), f32),
    )(x2, w1, b1, w2, b2, masks_col)

    eo, fv, cls, loss = pl.pallas_call(
        _k6_body,
        grid=(NT,),
        in_specs=[
            pl.BlockSpec((BT, D), lambda t: (t, 0)),
            pl.BlockSpec((1, D), lambda t: (0, 0)),
            pl.BlockSpec((1, D), lambda t: (0, 0)),
            pl.BlockSpec((D, D), lambda t: (0, 0)),
            pl.BlockSpec((1, D), lambda t: (0, 0)),
            pl.BlockSpec((8, 1), lambda t: (0, 0)),
            pl.BlockSpec((8, 1), lambda t: (0, 0)),
        ],
        out_specs=[
            pl.BlockSpec((BT, D), lambda t: (t, 0)),
            pl.BlockSpec((BATCH, D), lambda t: (0, 0)),
            pl.BlockSpec((BATCH, D), lambda t: (0, 0)),
            pl.BlockSpec((1, 1), lambda t: (0, 0)),
        ],
        out_shape=[
            jax.ShapeDtypeStruct((T, D), f32),
            jax.ShapeDtypeStruct((BATCH, D), f32),
            jax.ShapeDtypeStruct((BATCH, D), f32),
            jax.ShapeDtypeStruct((1, 1), f32),
        ],
    )(acc, row(p['ln3_g']), row(p['ln3_b']), p['cls_W'], row(p['cls_b']),
      me_sum, ce_sum)

    masks = masks_col[:, :NE].reshape(BATCH, SEQ, NE)
    expert_outputs = eo.reshape(BATCH, SEQ, D)
    return fv, cls, loss[0, 0], attn, expert_outputs, masks
